# baseline (speedup 1.0000x reference)
import functools

import jax
import jax.numpy as jnp
import numpy as np
from jax.experimental import pallas as pl
from jax.experimental.pallas import tpu as pltpu

_THR = np.float32(0.2)
_EPS = np.float32(1e-9)
_RB = 128
_MAXBLK = 28


def _top2(p):
    c = jax.lax.broadcasted_iota(jnp.int32, p.shape, 1)
    m1 = jnp.max(p, axis=1, keepdims=True)
    i1 = jnp.min(jnp.where(p >= m1, c, p.shape[1]), axis=1, keepdims=True)
    p2 = jnp.where(c == i1, jnp.float32(-1.0), p)
    m2 = jnp.max(p2, axis=1, keepdims=True)
    i2 = jnp.min(jnp.where(p2 >= m2, c, p.shape[1]), axis=1, keepdims=True)
    return m1, i1, m2, i2


def _softmax(l):
    e = jnp.exp(l - jnp.max(l, axis=1, keepdims=True))
    return e / jnp.sum(e, axis=1, keepdims=True)


def _router_body(eo, ei, rpad, x_ref, wg_ref, uo_ref, ui_ref, mean_ref,
                 std_ref, ltb_ref, ltp_ref, xh_ref, pmat_ref, cmat_ref,
                 cnt_ref):
    x = x_ref[...]
    xh = (x - mean_ref[...]) / std_ref[...]
    xh_ref[...] = xh.astype(jnp.bfloat16)
    logits = jnp.dot(xh, wg_ref[...], preferred_element_type=jnp.float32)

    po = _softmax(logits[:, 0:eo])
    g1, o1, g2, o2 = _top2(po)
    den = g1 + g2 + _EPS
    go1 = g1 / den
    go2 = g2 / den
    keep2 = uo_ref[...] < go2 / _THR

    qs = [_softmax(logits[:, eo + ei * e: eo + ei * (e + 1)]) for e in range(eo)]
    qb = jnp.zeros_like(qs[0])
    qc = jnp.zeros_like(qs[0])
    ub = jnp.zeros_like(g1)
    for e in range(eo):
        qb = qb + jnp.where(o1 == e, qs[e], 0.0)
        qc = qc + jnp.where(o2 == e, qs[e], 0.0)
        ub = ub + jnp.where(o1 == e, ui_ref[:, e:e + 1], 0.0)

    q1, i1, q2, i2 = _top2(qb)
    deni = q1 + q2 + _EPS
    gi1 = q1 / deni
    gi2 = q2 / deni
    keepi = ub < gi2 / _THR

    qm, j1, _, _ = _top2(qc)

    npair = eo * ei
    pk1 = o1 * ei + i1
    pk2 = o1 * ei + i2
    pk3 = o2 * ei + j1
    wk1 = go1 * gi1
    wk2 = jnp.where(keepi, go1 * gi2, 0.0)
    wk3 = jnp.where(keep2, go2 * (qm / (qm + _EPS)), 0.0)

    cp = jax.lax.broadcasted_iota(jnp.int32, (x.shape[0], npair), 1)
    act = ((cp == pk1).astype(jnp.float32)
           + jnp.where(keepi, (cp == pk2).astype(jnp.float32), 0.0)
           + jnp.where(keep2, (cp == pk3).astype(jnp.float32), 0.0))
    pos = jnp.dot(ltb_ref[...], act, preferred_element_type=jnp.float32)
    counts = jnp.sum(act, axis=0, keepdims=True)
    cnt_ref[...] = jnp.broadcast_to(counts, cnt_ref.shape)

    nblkp = jnp.floor((counts + np.float32(_RB - 1)) / np.float32(_RB))
    rowoff = np.float32(_RB) * jnp.dot(nblkp, ltp_ref[...],
                                       preferred_element_type=jnp.float32)
    destbase = rowoff + pos

    def sel(pk):
        return jnp.sum(jnp.where(cp == pk, destbase, 0.0), axis=1,
                       keepdims=True)

    d1 = sel(pk1)
    d2 = jnp.where(keepi, sel(pk2), -1.0)
    d3 = jnp.where(keep2, sel(pk3), -1.0)

    r = jax.lax.broadcasted_iota(jnp.int32, (x.shape[0], rpad), 1)
    rf = r.astype(jnp.float32)
    m1_ = (rf == d1).astype(jnp.float32)
    m2_ = (rf == d2).astype(jnp.float32)
    m3_ = (rf == d3).astype(jnp.float32)
    pmat_ref[...] = (m1_ + m2_ + m3_).astype(jnp.bfloat16)
    cmat_ref[...] = (wk1 * m1_ + wk2 * m2_ + wk3 * m3_).astype(jnp.bfloat16)


def _gather_body(p_ref, xh_ref, xd_ref):
    xd = jax.lax.dot_general(p_ref[...], xh_ref[...],
                             (((0,), (0,)), ((), ())),
                             preferred_element_type=jnp.float32)
    xd_ref[...] = xd.astype(jnp.bfloat16)


def _ffn_body(rb, d, pid_ref, nblk_ref, xd_ref, w1_ref, w2_ref, out_ref):
    h = pl.program_id(0)
    i = pl.program_id(1)
    sl = pl.ds(i * rb, rb)

    @pl.when(h == 0)
    def _init():
        out_ref[sl, :] = jnp.zeros((rb, d), jnp.float32)

    @pl.when(i < nblk_ref[0])
    def _work():
        w1b = w1_ref[0].astype(jnp.bfloat16)
        hid = jnp.dot(xd_ref[sl, :], w1b, preferred_element_type=jnp.float32)
        hid = jnp.maximum(hid, 0.0).astype(jnp.bfloat16)
        w2b = w2_ref[0].astype(jnp.bfloat16)
        out_ref[sl, :] += jnp.dot(hid, w2b, preferred_element_type=jnp.float32)


def _combine_body(c_ref, yd_ref, x_ref, ostd_ref, omean_ref, out_ref):
    moe = jnp.dot(c_ref[...], yd_ref[...].astype(jnp.bfloat16),
                  preferred_element_type=jnp.float32)
    out_ref[...] = x_ref[...] + moe * ostd_ref[...] + omean_ref[...]


def kernel(x, w_gate_outer, w_gate_inner, w1, w2, input_mean, input_std,
           output_mean, output_std):
    B, D = x.shape
    EO = w_gate_outer.shape[-1]
    EI = w_gate_inner.shape[-1]
    H = w1.shape[-1]
    NP = EO * EI
    HB = 1024
    NH = H // HB
    RPAD = _MAXBLK * _RB

    k1, k2 = jax.random.split(jax.random.key(42))
    u_out = jax.random.uniform(k1, (B, 1), dtype=jnp.float32)
    u_in = jnp.transpose(jax.random.uniform(k2, (EO, B, EI),
                                            dtype=jnp.float32)[:, :, 0])

    wg = jnp.concatenate(
        [w_gate_outer,
         jnp.transpose(w_gate_inner, (1, 0, 2)).reshape(D, EO * EI)], axis=1)
    ltb = jnp.tril(jnp.ones((B, B), jnp.float32), k=-1)
    ltp = jnp.triu(jnp.ones((NP, NP), jnp.float32), k=1)

    xh, pmat, cmat, cnt8 = pl.pallas_call(
        functools.partial(_router_body, EO, EI, RPAD),
        out_shape=(jax.ShapeDtypeStruct((B, D), jnp.bfloat16),
                   jax.ShapeDtypeStruct((B, RPAD), jnp.bfloat16),
                   jax.ShapeDtypeStruct((B, RPAD), jnp.bfloat16),
                   jax.ShapeDtypeStruct((8, NP), jnp.float32)),
    )(x, wg, u_out, u_in, input_mean.reshape(1, D), input_std.reshape(1, D),
      ltb, ltp)

    xd = pl.pallas_call(
        _gather_body,
        out_shape=jax.ShapeDtypeStruct((RPAD, D), jnp.bfloat16),
    )(pmat, xh)

    counts = cnt8[0].astype(jnp.int32)
    nblkp = (counts + (_RB - 1)) // _RB
    pid = jnp.repeat(jnp.arange(NP, dtype=jnp.int32), nblkp,
                     total_repeat_length=_MAXBLK)
    nblk = jnp.sum(nblkp, dtype=jnp.int32).reshape(1)

    w1f = w1.reshape(NP, D, H)
    w2f = w2.reshape(NP, H, D)

    yd = pl.pallas_call(
        functools.partial(_ffn_body, _RB, D),
        grid_spec=pltpu.PrefetchScalarGridSpec(
            num_scalar_prefetch=2,
            grid=(NH, _MAXBLK),
            in_specs=[
                pl.BlockSpec((RPAD, D), lambda h, i, pid, nblk: (0, 0)),
                pl.BlockSpec((1, D, HB), lambda h, i, pid, nblk: (pid[i], 0, h)),
                pl.BlockSpec((1, HB, D), lambda h, i, pid, nblk: (pid[i], h, 0)),
            ],
            out_specs=pl.BlockSpec((RPAD, D), lambda h, i, pid, nblk: (0, 0)),
        ),
        out_shape=jax.ShapeDtypeStruct((RPAD, D), jnp.float32),
        compiler_params=pltpu.CompilerParams(
            dimension_semantics=("arbitrary", "arbitrary")),
    )(pid, nblk, xd, w1f, w2f)

    out = pl.pallas_call(
        _combine_body,
        out_shape=jax.ShapeDtypeStruct((B, D), jnp.float32),
    )(cmat, yd, x, output_std.reshape(1, D), output_mean.reshape(1, D))
    return out


# speedup vs baseline: 1.2032x; 1.2032x over previous
import functools

import jax
import jax.numpy as jnp
from jax.experimental import pallas as pl
from jax.experimental.pallas import tpu as pltpu

import numpy as np

_THR = np.float32(0.2)
_EPS = np.float32(1e-9)


def _top2(p):
    c = jax.lax.broadcasted_iota(jnp.int32, p.shape, 1)
    m1 = jnp.max(p, axis=1, keepdims=True)
    i1 = jnp.min(jnp.where(p >= m1, c, p.shape[1]), axis=1, keepdims=True)
    p2 = jnp.where(c == i1, jnp.float32(-1.0), p)
    m2 = jnp.max(p2, axis=1, keepdims=True)
    i2 = jnp.min(jnp.where(p2 >= m2, c, p.shape[1]), axis=1, keepdims=True)
    return m1, i1, m2, i2


def _softmax(l):
    e = jnp.exp(l - jnp.max(l, axis=1, keepdims=True))
    return e / jnp.sum(e, axis=1, keepdims=True)


def _router_body(eo, ei, x_ref, wg_ref, uo_ref, ui_ref, mean_ref, std_ref,
                 xh_ref, w16_ref):
    x = x_ref[...]
    xh = (x - mean_ref[...]) / std_ref[...]
    xh_ref[...] = xh.astype(jnp.bfloat16)
    logits = jnp.dot(xh, wg_ref[...], preferred_element_type=jnp.float32)

    po = _softmax(logits[:, 0:eo])
    g1, o1, g2, o2 = _top2(po)
    den = g1 + g2 + _EPS
    go1 = g1 / den
    go2 = g2 / den
    keep2 = (uo_ref[...] < go2 / _THR).astype(jnp.float32)

    qs = [_softmax(logits[:, eo + ei * e: eo + ei * (e + 1)]) for e in range(eo)]
    zero = jnp.zeros_like(qs[0])
    qb = zero
    qc = zero
    ub = jnp.zeros_like(g1)
    for e in range(eo):
        qb = qb + jnp.where(o1 == e, qs[e], 0.0)
        qc = qc + jnp.where(o2 == e, qs[e], 0.0)
        ub = ub + jnp.where(o1 == e, ui_ref[:, e:e + 1], 0.0)

    q1, i1, q2, i2 = _top2(qb)
    deni = q1 + q2 + _EPS
    gi1 = q1 / deni
    gi2 = q2 / deni
    keep_i2 = (ub < gi2 / _THR).astype(jnp.float32)

    qm, j1, _, _ = _top2(qc)
    w3 = go2 * (qm / (qm + _EPS)) * keep2

    cp = jax.lax.broadcasted_iota(jnp.int32, (x.shape[0], eo * ei), 1)
    w16 = (jnp.where(cp == o1 * ei + i1, go1 * gi1, 0.0)
           + jnp.where(cp == o1 * ei + i2, keep_i2 * go1 * gi2, 0.0)
           + jnp.where(cp == o2 * ei + j1, w3, 0.0))
    w16_ref[...] = w16


def _ffn_body(np_, nh, x_ref, xh_ref, w1_ref, w2_ref, w16_ref, ostd_ref,
              omean_ref, out_ref):
    p = pl.program_id(0)
    h = pl.program_id(1)

    @pl.when((p == 0) & (h == 0))
    def _init():
        out_ref[...] = jnp.zeros_like(out_ref)

    w1b = w1_ref[0].astype(jnp.bfloat16)
    hid = jnp.dot(xh_ref[...], w1b, preferred_element_type=jnp.float32)
    hid = jnp.maximum(hid, 0.0)
    cp = jax.lax.broadcasted_iota(jnp.int32, w16_ref.shape, 1)
    wcol = jnp.sum(jnp.where(cp == p, w16_ref[...], 0.0), axis=1, keepdims=True)
    hid = (hid * wcol).astype(jnp.bfloat16)
    w2b = w2_ref[0].astype(jnp.bfloat16)
    out_ref[...] += jnp.dot(hid, w2b, preferred_element_type=jnp.float32)

    @pl.when((p == np_ - 1) & (h == nh - 1))
    def _fin():
        out_ref[...] = (x_ref[...] + out_ref[...] * ostd_ref[...]
                        + omean_ref[...])


def kernel(x, w_gate_outer, w_gate_inner, w1, w2, input_mean, input_std,
           output_mean, output_std):
    B, D = x.shape
    EO = w_gate_outer.shape[-1]
    EI = w_gate_inner.shape[-1]
    H = w1.shape[-1]
    NP = EO * EI
    HB = 2048
    NH = H // HB

    k1, k2 = jax.random.split(jax.random.key(42))
    u_out = jax.random.uniform(k1, (B, 1), dtype=jnp.float32)
    u_in = jnp.transpose(jax.random.uniform(k2, (EO, B, EI),
                                            dtype=jnp.float32)[:, :, 0])

    wg = jnp.concatenate(
        [w_gate_outer,
         jnp.transpose(w_gate_inner, (1, 0, 2)).reshape(D, EO * EI)], axis=1)

    xh, w16 = pl.pallas_call(
        functools.partial(_router_body, EO, EI),
        out_shape=(jax.ShapeDtypeStruct((B, D), jnp.bfloat16),
                   jax.ShapeDtypeStruct((B, NP), jnp.float32)),
    )(x, wg, u_out, u_in, input_mean.reshape(1, D), input_std.reshape(1, D))

    w1f = w1.reshape(NP, D, H)
    w2f = w2.reshape(NP, H, D)

    out = pl.pallas_call(
        functools.partial(_ffn_body, NP, NH),
        grid=(NP, NH),
        in_specs=[
            pl.BlockSpec((B, D), lambda p, h: (0, 0)),
            pl.BlockSpec((B, D), lambda p, h: (0, 0)),
            pl.BlockSpec((1, D, HB), lambda p, h: (p, 0, h)),
            pl.BlockSpec((1, HB, D), lambda p, h: (p, h, 0)),
            pl.BlockSpec((B, NP), lambda p, h: (0, 0)),
            pl.BlockSpec((1, D), lambda p, h: (0, 0)),
            pl.BlockSpec((1, D), lambda p, h: (0, 0)),
        ],
        out_specs=pl.BlockSpec((B, D), lambda p, h: (0, 0)),
        out_shape=jax.ShapeDtypeStruct((B, D), jnp.float32),
        compiler_params=pltpu.CompilerParams(
            dimension_semantics=("arbitrary", "arbitrary")),
    )(x, xh, w1f, w2f, w16, output_std.reshape(1, D),
      output_mean.reshape(1, D))
    return out
